# baseline (speedup 1.0000x reference)
import functools

import jax
import jax.numpy as jnp
from jax import lax
from jax.experimental import pallas as pl
from jax.experimental.pallas import tpu as pltpu
from jax.experimental.pallas import tpu_sc as plsc

D = 16
NC, NS = 2, 16
NW = NC * NS


def _pick_block_chunks(total_chunks: int) -> int:
    for c in range(96, 0, -1):
        if total_chunks % c == 0:
            return c
    return 1


def _make_sc_call(n: int):
    total_chunks = n // D
    bc = _pick_block_chunks(total_chunks)
    block_elems = bc * D
    total_blocks = total_chunks // bc
    mesh = plsc.VectorSubcoreMesh(
        core_axis_name="c", subcore_axis_name="s", num_cores=NC, num_subcores=NS
    )

    @functools.partial(
        pl.kernel,
        out_type=jax.ShapeDtypeStruct((n * D,), jnp.float32),
        mesh=mesh,
        scratch_types=[
            pltpu.VMEM((block_elems,), jnp.float32),
            pltpu.VMEM((block_elems,), jnp.int32),
            pltpu.VMEM((block_elems * D,), jnp.float32),
        ],
        compiler_params=pltpu.CompilerParams(needs_layout_passes=False),
    )
    def sc_encode(x_hbm, idx_hbm, out_hbm, xbuf, ibuf, obuf):
        wid = lax.axis_index("s") * NC + lax.axis_index("c")
        base = total_blocks // NW
        rem = total_blocks % NW
        nblk = base + jnp.where(wid < rem, 1, 0)
        first = wid * base + jnp.minimum(wid, rem)

        row16 = lax.iota(jnp.int32, D) * D
        ones = jnp.full((D,), 1.0, jnp.float32)
        zeros = jnp.zeros((D,), jnp.float32)

        def block_body(i, carry):
            eoff = (first + i) * block_elems
            pltpu.sync_copy(x_hbm.at[pl.ds(eoff, block_elems)], xbuf)
            pltpu.sync_copy(idx_hbm.at[pl.ds(eoff, block_elems)], ibuf)

            for c in range(bc):
                xv = xbuf[pl.ds(c * D, D)]
                iv = ibuf[pl.ds(c * D, D)]
                bvec = row16 + c * (D * D)
                for t in range(D):
                    col = jnp.where(iv > t, ones, zeros)
                    plsc.store_scatter(obuf, [bvec + t], col)
                plsc.store_scatter(obuf, [bvec + iv], xv)
            pltpu.sync_copy(obuf, out_hbm.at[pl.ds(eoff * D, block_elems * D)])
            return carry

        lax.fori_loop(0, nblk, block_body, 0, unroll=False)

    return sc_encode


def kernel(x, indices):
    n_rows, n_feat = x.shape
    n = n_rows * n_feat
    enc = _make_sc_call(n)(x.reshape(n), indices.reshape(n))
    return enc.reshape(n_rows, n_feat, D)


# speedup vs baseline: 30.9653x; 30.9653x over previous
import jax
import jax.numpy as jnp
from jax import lax
from jax.experimental import pallas as pl

D = 16


def _body(x_ref, i_ref, o_ref):
    iv = i_ref[...]
    xv = x_ref[...]
    ib = iv[:, None, :]
    xb = xv[:, None, :]
    t = lax.broadcasted_iota(jnp.int32, (iv.shape[0], D, iv.shape[1]), 1)
    one = jnp.float32(1.0)
    zero = jnp.float32(0.0)
    o_ref[...] = jnp.where(t < ib, one, jnp.where(t == ib, xb, zero))


def kernel(x, indices):
    nr, nf = x.shape
    xT = x.transpose(1, 0)
    iT = indices.transpose(1, 0)
    C = 4096
    grid = (nr + C - 1) // C
    out = pl.pallas_call(
        _body,
        grid=(grid,),
        in_specs=[
            pl.BlockSpec((nf, C), lambda i: (0, i)),
            pl.BlockSpec((nf, C), lambda i: (0, i)),
        ],
        out_specs=pl.BlockSpec((nf, D, C), lambda i: (0, 0, i)),
        out_shape=jax.ShapeDtypeStruct((nf, D, nr), jnp.float32),
    )(xT, iT)
    return out.transpose(2, 0, 1)
